# baseline (speedup 1.0000x reference)
import functools

import jax
import jax.numpy as jnp
from jax import lax
from jax.experimental import pallas as pl
from jax.experimental.pallas import tpu as pltpu
from jax.experimental.pallas import tpu_sc as plsc

NUM_POS = 4096
NUM_DIM = 1024
BATCH = 4

_NC = 2
_NS = 16
_NW = _NC * _NS
_ROWS_PER_W = NUM_POS // _NW
_HALF = _ROWS_PER_W // 2

_mesh = plsc.VectorSubcoreMesh(core_axis_name="c", subcore_axis_name="s")


@functools.partial(
    pl.kernel,
    mesh=_mesh,
    out_type=jax.ShapeDtypeStruct((BATCH, NUM_POS, NUM_DIM), jnp.float32),
    scratch_types=[
        pltpu.VMEM_SHARED((_NS * _ROWS_PER_W, NUM_DIM), jnp.float32),
        pltpu.SemaphoreType.DMA,
        pltpu.SemaphoreType.DMA,
    ],
)
def _broadcast_table(w_hbm, out_hbm, shared, rsem, wsem):
    sid = lax.axis_index("s")
    wid = sid * _NC + lax.axis_index("c")
    base = wid * _ROWS_PER_W
    sbase = sid * _ROWS_PER_W

    reads = [
        pltpu.async_copy(
            w_hbm.at[pl.ds(base + h * _HALF, _HALF)],
            shared.at[pl.ds(sbase + h * _HALF, _HALF)],
            rsem,
        )
        for h in range(2)
    ]
    writes = []
    for h in range(2):
        reads[h].wait()
        for b in range(BATCH):
            writes.append(
                pltpu.async_copy(
                    shared.at[pl.ds(sbase + h * _HALF, _HALF)],
                    out_hbm.at[b, pl.ds(base + h * _HALF, _HALF)],
                    wsem,
                )
            )
    for wr in writes:
        wr.wait()


def kernel(x, w):
    del x
    return _broadcast_table(w)


# speedup vs baseline: 1.2222x; 1.2222x over previous
import functools

import jax
import jax.numpy as jnp
from jax import lax
from jax.experimental import pallas as pl
from jax.experimental.pallas import tpu as pltpu
from jax.experimental.pallas import tpu_sc as plsc

NUM_POS = 4096
NUM_DIM = 1024
BATCH = 4

_NC = 2
_NS = 16
_NW = _NC * _NS
_ROWS_PER_W = NUM_POS // _NW
_CHUNK = 32
_NCH = _ROWS_PER_W // _CHUNK
_NBUF = 3

_mesh = plsc.VectorSubcoreMesh(core_axis_name="c", subcore_axis_name="s")


@functools.partial(
    pl.kernel,
    mesh=_mesh,
    out_type=jax.ShapeDtypeStruct((BATCH, NUM_POS, NUM_DIM), jnp.float32),
    scratch_types=(
        [pltpu.VMEM((_CHUNK, NUM_DIM), jnp.float32) for _ in range(_NBUF)]
        + [pltpu.SemaphoreType.DMA, pltpu.SemaphoreType.DMA]
    ),
)
def _broadcast_table(w_hbm, out_hbm, buf0, buf1, buf2, rsem, wsem):
    bufs = (buf0, buf1, buf2)
    wid = lax.axis_index("s") * _NC + lax.axis_index("c")
    base = wid * _ROWS_PER_W

    reads = {}
    for c in range(_NBUF):
        reads[c] = pltpu.async_copy(
            w_hbm.at[pl.ds(base + c * _CHUNK, _CHUNK)], bufs[c % _NBUF], rsem)

    writes = []
    for c in range(_NCH):
        reads[c].wait()
        for b in range(BATCH):
            writes.append(pltpu.async_copy(
                bufs[c % _NBUF],
                out_hbm.at[b, pl.ds(base + c * _CHUNK, _CHUNK)],
                wsem,
            ))
        nxt = c + _NBUF
        if nxt < _NCH:
            for _ in range(BATCH):
                writes.pop(0).wait()
            reads[nxt] = pltpu.async_copy(
                w_hbm.at[pl.ds(base + nxt * _CHUNK, _CHUNK)],
                bufs[nxt % _NBUF], rsem)
    for wr in writes:
        wr.wait()


def kernel(x, w):
    del x
    return _broadcast_table(w)
